# baseline (speedup 1.0000x reference)
import jax
import jax.numpy as jnp
from jax.experimental import pallas as pl
from jax.experimental.pallas import tpu as pltpu

_BLOCK = 2 * 1024 * 1024


def _eta_body(em_ref, x_ref, o_ref):
    e = em_ref[0]
    m = em_ref[1]
    x = x_ref[...]
    o_ref[...] = jnp.where(x > e, x * (m * e), 1.0 - x / e)


def kernel(loss, eta, mask):
    n = loss.shape[0]
    em = jnp.concatenate([eta, mask])
    out = pl.pallas_call(
        _eta_body,
        grid=(n // _BLOCK,),
        in_specs=[
            pl.BlockSpec(memory_space=pltpu.SMEM),
            pl.BlockSpec((_BLOCK,), lambda i: (i,)),
        ],
        out_specs=pl.BlockSpec((_BLOCK,), lambda i: (i,)),
        out_shape=jax.ShapeDtypeStruct((n,), jnp.float32),
        compiler_params=pltpu.CompilerParams(
            dimension_semantics=("parallel",),
            vmem_limit_bytes=48 * 1024 * 1024,
        ),
    )(em, loss)
    return out


# speedup vs baseline: 1.0015x; 1.0015x over previous
import jax
import jax.numpy as jnp
from jax.experimental import pallas as pl
from jax.experimental.pallas import tpu as pltpu

_BLOCK = 2 * 1024 * 1024


def _eta_body(eta_ref, mask_ref, x_ref, o_ref):
    e = eta_ref[0]
    m = mask_ref[0]
    x = x_ref[...]
    o_ref[...] = jnp.where(x > e, x * (m * e), 1.0 - x / e)


def kernel(loss, eta, mask):
    n = loss.shape[0]
    out = pl.pallas_call(
        _eta_body,
        grid=(n // _BLOCK,),
        in_specs=[
            pl.BlockSpec(memory_space=pltpu.SMEM),
            pl.BlockSpec(memory_space=pltpu.SMEM),
            pl.BlockSpec((_BLOCK,), lambda i: (i,)),
        ],
        out_specs=pl.BlockSpec((_BLOCK,), lambda i: (i,)),
        out_shape=jax.ShapeDtypeStruct((n,), jnp.float32),
        compiler_params=pltpu.CompilerParams(
            dimension_semantics=("parallel",),
            vmem_limit_bytes=48 * 1024 * 1024,
        ),
    )(eta, mask, loss)
    return out
